# baseline (speedup 1.0000x reference)
import functools

import jax
import jax.numpy as jnp
from jax.experimental import pallas as pl
from jax.experimental.pallas import tpu as pltpu
from jax.experimental.pallas import tpu_sc as plsc

TOPK = 2


def _sc_row_gather(table, idx):
    B = idx.shape[0]
    _, D = table.shape
    info = plsc.get_sparse_core_info()
    NW = info.num_cores * info.num_subcores
    b_per_w = B // NW
    CH = min(64, b_per_w)
    n_ch = b_per_w // CH
    mesh = plsc.VectorSubcoreMesh(core_axis_name="c", subcore_axis_name="s")

    @functools.partial(
        pl.kernel, mesh=mesh,
        out_type=jax.ShapeDtypeStruct((B, D), table.dtype),
        scratch_types=[
            pltpu.VMEM((2, CH), jnp.int32),
            pltpu.VMEM((2, CH, D), table.dtype),
            pltpu.SemaphoreType.DMA,
            pltpu.SemaphoreType.DMA,
        ],
    )
    def k(table_hbm, idx_hbm, out_hbm, idx_v, rows_v, sem0, sem1):
        wid = jax.lax.axis_index("s") * info.num_cores + jax.lax.axis_index("c")
        base = wid * b_per_w
        sems = (sem0, sem1)
        pltpu.sync_copy(idx_hbm.at[pl.ds(base, CH)], idx_v.at[0])
        pending = [pltpu.async_copy(table_hbm.at[idx_v.at[0]], rows_v.at[0],
                                    sems[0])]
        for c in range(n_ch):
            cur = c % 2
            if c + 1 < n_ch:
                nxt = (c + 1) % 2
                o2 = base + (c + 1) * CH
                pltpu.sync_copy(idx_hbm.at[pl.ds(o2, CH)], idx_v.at[nxt])
                pending.append(pltpu.async_copy(table_hbm.at[idx_v.at[nxt]],
                                                rows_v.at[nxt], sems[nxt]))
            pending[c].wait()
            pltpu.sync_copy(rows_v.at[cur], out_hbm.at[pl.ds(base + c * CH, CH)])

    return k(table, idx)


def _gate_kernel(x_ref, gwT_ref, gb_ref, i1_ref, i2_ref, w1_ref, w2_ref):
    xb = x_ref[...]
    gb = gb_ref[0, :]
    logits = jnp.dot(xb, gwT_ref[...], preferred_element_type=jnp.float32)
    logits = logits + gb[None, :]
    orig = jax.nn.sigmoid(logits)
    scores = orig + gb[None, :]
    E = scores.shape[1]
    iotaE = jax.lax.broadcasted_iota(jnp.int32, scores.shape, 1)
    m1 = jnp.max(scores, axis=1, keepdims=True)
    i1 = jnp.min(jnp.where(scores == m1, iotaE, E), axis=1)
    oh1 = iotaE == i1[:, None]
    v1 = jnp.sum(jnp.where(oh1, orig, 0.0), axis=1)
    scores2 = jnp.where(oh1, -jnp.inf, scores)
    m2 = jnp.max(scores2, axis=1, keepdims=True)
    i2 = jnp.min(jnp.where(scores2 == m2, iotaE, E), axis=1)
    oh2 = iotaE == i2[:, None]
    v2 = jnp.sum(jnp.where(oh2, orig, 0.0), axis=1)
    s = v1 + v2
    i1_ref[0, 0, :] = i1.astype(jnp.int32)
    i2_ref[0, 0, :] = i2.astype(jnp.int32)
    w1_ref[0, 0, :] = v1 / s
    w2_ref[0, 0, :] = v2 / s


def _moe_kernel(bids_ref, gids_ref, rs_ref, re_ref, init_ref,
                xs_ref, w_ref, W1_ref, B1_ref, W3_ref, B3_ref, W2_ref, B2_ref,
                out_ref, *, blk):
    t = pl.program_id(0)
    rs = rs_ref[t]
    re_ = re_ref[t]

    @pl.when(re_ > rs)
    def _():
        xb = xs_ref[...]
        gr = bids_ref[t] * blk + jax.lax.broadcasted_iota(jnp.int32, (blk, 1), 0)
        mask = (gr >= rs) & (gr < re_)
        wv = w_ref[0, 0, :].reshape(blk, 1) * mask.astype(jnp.float32)
        h1 = jnp.dot(xb, W1_ref[0].astype(jnp.bfloat16),
                     preferred_element_type=jnp.float32) + B1_ref[0]
        h3 = jnp.dot(xb, W3_ref[0].astype(jnp.bfloat16),
                     preferred_element_type=jnp.float32) + B3_ref[0]
        h = (jax.nn.silu(h1) * h3 * wv).astype(jnp.bfloat16)
        contrib = jnp.dot(h, W2_ref[0].astype(jnp.bfloat16),
                          preferred_element_type=jnp.float32)
        contrib = contrib + wv * B2_ref[0]

        @pl.when(init_ref[t] == 1)
        def _():
            out_ref[...] = contrib

        @pl.when(init_ref[t] == 0)
        def _():
            out_ref[...] += contrib


def _shared_kernel(x_ref, sw1_ref, sb1_ref, sw3_ref, sb3_ref,
                   sw2_ref, sb2_ref, o_ref):
    xb = x_ref[...]
    xb16 = xb.astype(jnp.bfloat16)
    h1 = jnp.dot(xb16, sw1_ref[...].astype(jnp.bfloat16),
                 preferred_element_type=jnp.float32) + sb1_ref[0, :][None, :]
    h3 = jnp.dot(xb16, sw3_ref[...].astype(jnp.bfloat16),
                 preferred_element_type=jnp.float32) + sb3_ref[0, :][None, :]
    h = (jax.nn.silu(h1) * h3).astype(jnp.bfloat16)
    z = jnp.dot(h, sw2_ref[...].astype(jnp.bfloat16),
                preferred_element_type=jnp.float32) + sb2_ref[0, :][None, :]
    o_ref[...] = z + xb


def _combine_kernel(zx_ref, y2_ref, o_ref):
    sb, d = o_ref.shape
    y2b = y2_ref[...].reshape(sb, 2, d).astype(jnp.float32)
    o_ref[...] = zx_ref[...] + y2b[:, 0, :] + y2b[:, 1, :]


def kernel(x, gate_w, gate_b, W1, B1, W2, B2, W3, B3, SW1, SB1, SW2, SB2, SW3, SB3):
    TOK, DIM = x.shape
    E, _, INTER = W1.shape
    TOKF = TOK * TOPK
    BLK = 256
    NB = TOKF // BLK
    NT = NB + E
    GB = min(1024, TOK)
    SB = min(512, TOK)

    n_gb = TOK // GB
    gwT = gate_w.T
    gb2 = gate_b.reshape(1, E)
    i1, i2, w1, w2 = pl.pallas_call(
        _gate_kernel,
        grid=(n_gb,),
        in_specs=[
            pl.BlockSpec((GB, DIM), lambda i: (i, 0)),
            pl.BlockSpec((DIM, E), lambda i: (0, 0)),
            pl.BlockSpec((1, E), lambda i: (0, 0)),
        ],
        out_specs=[
            pl.BlockSpec((1, 1, GB), lambda i: (i, 0, 0)),
            pl.BlockSpec((1, 1, GB), lambda i: (i, 0, 0)),
            pl.BlockSpec((1, 1, GB), lambda i: (i, 0, 0)),
            pl.BlockSpec((1, 1, GB), lambda i: (i, 0, 0)),
        ],
        out_shape=[
            jax.ShapeDtypeStruct((n_gb, 1, GB), jnp.int32),
            jax.ShapeDtypeStruct((n_gb, 1, GB), jnp.int32),
            jax.ShapeDtypeStruct((n_gb, 1, GB), jnp.float32),
            jax.ShapeDtypeStruct((n_gb, 1, GB), jnp.float32),
        ],
    )(x, gwT, gb2)
    i1 = i1.reshape(TOK)
    i2 = i2.reshape(TOK)
    w1 = w1.reshape(TOK)
    w2 = w2.reshape(TOK)

    e_flat = jnp.stack([i1, i2], axis=1).reshape(-1)
    w_flat = jnp.stack([w1, w2], axis=1).reshape(-1)

    flat_ids = jnp.arange(TOKF, dtype=jnp.int32)
    order = jnp.argsort(e_flat).astype(jnp.int32)
    sorted_tok = order // TOPK
    sorted_w = jnp.take(w_flat, order)
    del flat_ids
    pos = jnp.argsort(order).astype(jnp.int32)
    counts = jnp.bincount(e_flat, length=E).astype(jnp.int32)
    off = jnp.concatenate([jnp.zeros(1, jnp.int32),
                           jnp.cumsum(counts)]).astype(jnp.int32)
    xs = jnp.take(x.astype(jnp.bfloat16), sorted_tok, axis=0)

    fb = off[:-1] // BLK
    lb = jnp.where(counts > 0, (off[1:] - 1) // BLK, 0)
    tiles_per = jnp.where(counts > 0, lb - fb + 1, 0).astype(jnp.int32)
    tile_off = jnp.concatenate([jnp.zeros(1, jnp.int32),
                                jnp.cumsum(tiles_per)]).astype(jnp.int32)
    T = tile_off[-1]
    gids = jnp.repeat(jnp.arange(E, dtype=jnp.int32), tiles_per,
                      total_repeat_length=NT)
    valid = jnp.arange(NT, dtype=jnp.int32) < T
    last_gid = jnp.take(gids, T - 1)
    gids = jnp.where(valid, gids, last_gid)
    j = jnp.arange(NT, dtype=jnp.int32) - tile_off[gids]
    bids = jnp.where(valid, fb[gids] + j, NB - 1).astype(jnp.int32)
    row_start = jnp.where(valid, off[gids], 0).astype(jnp.int32)
    row_end = jnp.where(valid, off[gids + 1], 0).astype(jnp.int32)
    init = jnp.concatenate([jnp.ones(1, jnp.int32),
                            (bids[1:] != bids[:-1]).astype(jnp.int32)])

    sw3d = sorted_w.reshape(NB, 1, BLK)

    grid_spec = pltpu.PrefetchScalarGridSpec(
        num_scalar_prefetch=5,
        grid=(NT,),
        in_specs=[
            pl.BlockSpec((BLK, DIM), lambda t, b, g, rs, re, it: (b[t], 0)),
            pl.BlockSpec((1, 1, BLK), lambda t, b, g, rs, re, it: (b[t], 0, 0)),
            pl.BlockSpec((1, DIM, INTER), lambda t, b, g, rs, re, it: (g[t], 0, 0)),
            pl.BlockSpec((1, 1, INTER), lambda t, b, g, rs, re, it: (g[t], 0, 0)),
            pl.BlockSpec((1, DIM, INTER), lambda t, b, g, rs, re, it: (g[t], 0, 0)),
            pl.BlockSpec((1, 1, INTER), lambda t, b, g, rs, re, it: (g[t], 0, 0)),
            pl.BlockSpec((1, INTER, DIM), lambda t, b, g, rs, re, it: (g[t], 0, 0)),
            pl.BlockSpec((1, 1, DIM), lambda t, b, g, rs, re, it: (g[t], 0, 0)),
        ],
        out_specs=pl.BlockSpec((BLK, DIM), lambda t, b, g, rs, re, it: (b[t], 0)),
    )
    outs = pl.pallas_call(
        functools.partial(_moe_kernel, blk=BLK),
        grid_spec=grid_spec,
        out_shape=jax.ShapeDtypeStruct((TOKF, DIM), jnp.float32),
    )(bids, gids, row_start, row_end, init,
      xs, sw3d, W1, B1.reshape(E, 1, INTER), W3, B3.reshape(E, 1, INTER),
      W2, B2.reshape(E, 1, DIM))

    y2 = _sc_row_gather(outs, pos)

    n_sb = TOK // SB
    zx = pl.pallas_call(
        _shared_kernel,
        grid=(n_sb,),
        in_specs=[
            pl.BlockSpec((SB, DIM), lambda i: (i, 0)),
            pl.BlockSpec((DIM, SW1.shape[1]), lambda i: (0, 0)),
            pl.BlockSpec((1, SW1.shape[1]), lambda i: (0, 0)),
            pl.BlockSpec((DIM, SW1.shape[1]), lambda i: (0, 0)),
            pl.BlockSpec((1, SW1.shape[1]), lambda i: (0, 0)),
            pl.BlockSpec((SW1.shape[1], DIM), lambda i: (0, 0)),
            pl.BlockSpec((1, DIM), lambda i: (0, 0)),
        ],
        out_specs=pl.BlockSpec((SB, DIM), lambda i: (i, 0)),
        out_shape=jax.ShapeDtypeStruct((TOK, DIM), jnp.float32),
    )(x, SW1, SB1.reshape(1, -1), SW3, SB3.reshape(1, -1),
      SW2, SB2.reshape(1, -1))

    out = pl.pallas_call(
        _combine_kernel,
        grid=(n_sb,),
        in_specs=[
            pl.BlockSpec((SB, DIM), lambda i: (i, 0)),
            pl.BlockSpec((TOPK * SB, DIM), lambda i: (i, 0)),
        ],
        out_specs=pl.BlockSpec((SB, DIM), lambda i: (i, 0)),
        out_shape=jax.ShapeDtypeStruct((TOK, DIM), jnp.float32),
    )(zx, y2)
    return out


# speedup vs baseline: 1.2381x; 1.2381x over previous
import functools

import jax
import jax.numpy as jnp
from jax.experimental import pallas as pl
from jax.experimental.pallas import tpu as pltpu
from jax.experimental.pallas import tpu_sc as plsc

TOPK = 2


def _sc_row_gather(table, idx):
    B = idx.shape[0]
    _, D = table.shape
    info = plsc.get_sparse_core_info()
    NW = info.num_cores * info.num_subcores
    b_per_w = B // NW
    CH = min(64, b_per_w)
    n_ch = b_per_w // CH
    mesh = plsc.VectorSubcoreMesh(core_axis_name="c", subcore_axis_name="s")

    @functools.partial(
        pl.kernel, mesh=mesh,
        out_type=jax.ShapeDtypeStruct((B, D), table.dtype),
        scratch_types=[
            pltpu.VMEM((2, CH), jnp.int32),
            pltpu.VMEM((2, CH, D), table.dtype),
            pltpu.SemaphoreType.DMA,
            pltpu.SemaphoreType.DMA,
        ],
    )
    def k(table_hbm, idx_hbm, out_hbm, idx_v, rows_v, sem0, sem1):
        wid = jax.lax.axis_index("s") * info.num_cores + jax.lax.axis_index("c")
        base = wid * b_per_w
        sems = (sem0, sem1)
        pltpu.sync_copy(idx_hbm.at[pl.ds(base, CH)], idx_v.at[0])
        pending = [pltpu.async_copy(table_hbm.at[idx_v.at[0]], rows_v.at[0],
                                    sems[0])]
        for c in range(n_ch):
            cur = c % 2
            if c + 1 < n_ch:
                nxt = (c + 1) % 2
                o2 = base + (c + 1) * CH
                pltpu.sync_copy(idx_hbm.at[pl.ds(o2, CH)], idx_v.at[nxt])
                pending.append(pltpu.async_copy(table_hbm.at[idx_v.at[nxt]],
                                                rows_v.at[nxt], sems[nxt]))
            pending[c].wait()
            pltpu.sync_copy(rows_v.at[cur], out_hbm.at[pl.ds(base + c * CH, CH)])

    return k(table, idx)


def _gate_kernel(x_ref, gwT_ref, gb_ref, i1_ref, i2_ref, w1_ref, w2_ref):
    xb = x_ref[...]
    gb = gb_ref[0, :]
    logits = jnp.dot(xb, gwT_ref[...], preferred_element_type=jnp.float32)
    logits = logits + gb[None, :]
    orig = jax.nn.sigmoid(logits)
    scores = orig + gb[None, :]
    E = scores.shape[1]
    iotaE = jax.lax.broadcasted_iota(jnp.int32, scores.shape, 1)
    m1 = jnp.max(scores, axis=1, keepdims=True)
    i1 = jnp.min(jnp.where(scores == m1, iotaE, E), axis=1)
    oh1 = iotaE == i1[:, None]
    v1 = jnp.sum(jnp.where(oh1, orig, 0.0), axis=1)
    scores2 = jnp.where(oh1, -jnp.inf, scores)
    m2 = jnp.max(scores2, axis=1, keepdims=True)
    i2 = jnp.min(jnp.where(scores2 == m2, iotaE, E), axis=1)
    oh2 = iotaE == i2[:, None]
    v2 = jnp.sum(jnp.where(oh2, orig, 0.0), axis=1)
    s = v1 + v2
    i1_ref[0, 0, :] = i1.astype(jnp.int32)
    i2_ref[0, 0, :] = i2.astype(jnp.int32)
    w1_ref[0, 0, :] = v1 / s
    w2_ref[0, 0, :] = v2 / s


def _moe_kernel(bids_ref, gids_ref, rs_ref, re_ref, init_ref,
                xs_ref, w_ref, W1_ref, B1_ref, W3_ref, B3_ref, W2_ref, B2_ref,
                out_ref, *, blk):
    t = pl.program_id(0)
    rs = rs_ref[t]
    re_ = re_ref[t]

    @pl.when(re_ > rs)
    def _():
        xb = xs_ref[...].astype(jnp.bfloat16)
        gr = bids_ref[t] * blk + jax.lax.broadcasted_iota(jnp.int32, (blk, 1), 0)
        mask = (gr >= rs) & (gr < re_)
        wv = w_ref[0, 0, :].reshape(blk, 1) * mask.astype(jnp.float32)
        h1 = jnp.dot(xb, W1_ref[0].astype(jnp.bfloat16),
                     preferred_element_type=jnp.float32) + B1_ref[0]
        h3 = jnp.dot(xb, W3_ref[0].astype(jnp.bfloat16),
                     preferred_element_type=jnp.float32) + B3_ref[0]
        h = (jax.nn.silu(h1) * h3 * wv).astype(jnp.bfloat16)
        contrib = jnp.dot(h, W2_ref[0].astype(jnp.bfloat16),
                          preferred_element_type=jnp.float32)
        contrib = contrib + wv * B2_ref[0]

        @pl.when(init_ref[t] == 1)
        def _():
            out_ref[...] = contrib

        @pl.when(init_ref[t] == 0)
        def _():
            out_ref[...] += contrib


def _shared_kernel(x_ref, sw1_ref, sb1_ref, sw3_ref, sb3_ref,
                   sw2_ref, sb2_ref, o_ref):
    xb = x_ref[...]
    xb16 = xb.astype(jnp.bfloat16)
    h1 = jnp.dot(xb16, sw1_ref[...].astype(jnp.bfloat16),
                 preferred_element_type=jnp.float32) + sb1_ref[0, :][None, :]
    h3 = jnp.dot(xb16, sw3_ref[...].astype(jnp.bfloat16),
                 preferred_element_type=jnp.float32) + sb3_ref[0, :][None, :]
    h = (jax.nn.silu(h1) * h3).astype(jnp.bfloat16)
    z = jnp.dot(h, sw2_ref[...].astype(jnp.bfloat16),
                preferred_element_type=jnp.float32) + sb2_ref[0, :][None, :]
    o_ref[...] = z + xb


def _combine_kernel(zx_ref, y2_ref, o_ref):
    sb, d = o_ref.shape
    y2b = y2_ref[...].reshape(sb, 2, d).astype(jnp.float32)
    o_ref[...] = zx_ref[...] + y2b[:, 0, :] + y2b[:, 1, :]


def kernel(x, gate_w, gate_b, W1, B1, W2, B2, W3, B3, SW1, SB1, SW2, SB2, SW3, SB3):
    TOK, DIM = x.shape
    E, _, INTER = W1.shape
    TOKF = TOK * TOPK
    BLK = 256
    NB = TOKF // BLK
    NT = NB + E
    GB = min(1024, TOK)
    SB = min(512, TOK)

    n_gb = TOK // GB
    gwT = gate_w.T
    gb2 = gate_b.reshape(1, E)
    i1, i2, w1, w2 = pl.pallas_call(
        _gate_kernel,
        grid=(n_gb,),
        in_specs=[
            pl.BlockSpec((GB, DIM), lambda i: (i, 0)),
            pl.BlockSpec((DIM, E), lambda i: (0, 0)),
            pl.BlockSpec((1, E), lambda i: (0, 0)),
        ],
        out_specs=[
            pl.BlockSpec((1, 1, GB), lambda i: (i, 0, 0)),
            pl.BlockSpec((1, 1, GB), lambda i: (i, 0, 0)),
            pl.BlockSpec((1, 1, GB), lambda i: (i, 0, 0)),
            pl.BlockSpec((1, 1, GB), lambda i: (i, 0, 0)),
        ],
        out_shape=[
            jax.ShapeDtypeStruct((n_gb, 1, GB), jnp.int32),
            jax.ShapeDtypeStruct((n_gb, 1, GB), jnp.int32),
            jax.ShapeDtypeStruct((n_gb, 1, GB), jnp.float32),
            jax.ShapeDtypeStruct((n_gb, 1, GB), jnp.float32),
        ],
    )(x, gwT, gb2)
    i1 = i1.reshape(TOK)
    i2 = i2.reshape(TOK)
    w1 = w1.reshape(TOK)
    w2 = w2.reshape(TOK)

    e_flat = jnp.stack([i1, i2], axis=1).reshape(-1)
    w_flat = jnp.stack([w1, w2], axis=1).reshape(-1)

    flat_ids = jnp.arange(TOKF, dtype=jnp.int32)
    order = jnp.argsort(e_flat).astype(jnp.int32)
    sorted_tok = order // TOPK
    sorted_w = jnp.take(w_flat, order)
    del flat_ids
    pos = jnp.argsort(order).astype(jnp.int32)
    counts = jnp.bincount(e_flat, length=E).astype(jnp.int32)
    off = jnp.concatenate([jnp.zeros(1, jnp.int32),
                           jnp.cumsum(counts)]).astype(jnp.int32)
    xs = _sc_row_gather(x, sorted_tok)

    fb = off[:-1] // BLK
    lb = jnp.where(counts > 0, (off[1:] - 1) // BLK, 0)
    tiles_per = jnp.where(counts > 0, lb - fb + 1, 0).astype(jnp.int32)
    tile_off = jnp.concatenate([jnp.zeros(1, jnp.int32),
                                jnp.cumsum(tiles_per)]).astype(jnp.int32)
    T = tile_off[-1]
    gids = jnp.repeat(jnp.arange(E, dtype=jnp.int32), tiles_per,
                      total_repeat_length=NT)
    valid = jnp.arange(NT, dtype=jnp.int32) < T
    last_gid = jnp.take(gids, T - 1)
    gids = jnp.where(valid, gids, last_gid)
    j = jnp.arange(NT, dtype=jnp.int32) - tile_off[gids]
    bids = jnp.where(valid, fb[gids] + j, NB - 1).astype(jnp.int32)
    row_start = jnp.where(valid, off[gids], 0).astype(jnp.int32)
    row_end = jnp.where(valid, off[gids + 1], 0).astype(jnp.int32)
    init = jnp.concatenate([jnp.ones(1, jnp.int32),
                            (bids[1:] != bids[:-1]).astype(jnp.int32)])

    sw3d = sorted_w.reshape(NB, 1, BLK)

    grid_spec = pltpu.PrefetchScalarGridSpec(
        num_scalar_prefetch=5,
        grid=(NT,),
        in_specs=[
            pl.BlockSpec((BLK, DIM), lambda t, b, g, rs, re, it: (b[t], 0)),
            pl.BlockSpec((1, 1, BLK), lambda t, b, g, rs, re, it: (b[t], 0, 0)),
            pl.BlockSpec((1, DIM, INTER), lambda t, b, g, rs, re, it: (g[t], 0, 0)),
            pl.BlockSpec((1, 1, INTER), lambda t, b, g, rs, re, it: (g[t], 0, 0)),
            pl.BlockSpec((1, DIM, INTER), lambda t, b, g, rs, re, it: (g[t], 0, 0)),
            pl.BlockSpec((1, 1, INTER), lambda t, b, g, rs, re, it: (g[t], 0, 0)),
            pl.BlockSpec((1, INTER, DIM), lambda t, b, g, rs, re, it: (g[t], 0, 0)),
            pl.BlockSpec((1, 1, DIM), lambda t, b, g, rs, re, it: (g[t], 0, 0)),
        ],
        out_specs=pl.BlockSpec((BLK, DIM), lambda t, b, g, rs, re, it: (b[t], 0)),
    )
    outs = pl.pallas_call(
        functools.partial(_moe_kernel, blk=BLK),
        grid_spec=grid_spec,
        out_shape=jax.ShapeDtypeStruct((TOKF, DIM), jnp.float32),
    )(bids, gids, row_start, row_end, init,
      xs, sw3d, W1, B1.reshape(E, 1, INTER), W3, B3.reshape(E, 1, INTER),
      W2, B2.reshape(E, 1, DIM))

    y2 = _sc_row_gather(outs, pos)

    n_sb = TOK // SB
    zx = pl.pallas_call(
        _shared_kernel,
        grid=(n_sb,),
        in_specs=[
            pl.BlockSpec((SB, DIM), lambda i: (i, 0)),
            pl.BlockSpec((DIM, SW1.shape[1]), lambda i: (0, 0)),
            pl.BlockSpec((1, SW1.shape[1]), lambda i: (0, 0)),
            pl.BlockSpec((DIM, SW1.shape[1]), lambda i: (0, 0)),
            pl.BlockSpec((1, SW1.shape[1]), lambda i: (0, 0)),
            pl.BlockSpec((SW1.shape[1], DIM), lambda i: (0, 0)),
            pl.BlockSpec((1, DIM), lambda i: (0, 0)),
        ],
        out_specs=pl.BlockSpec((SB, DIM), lambda i: (i, 0)),
        out_shape=jax.ShapeDtypeStruct((TOK, DIM), jnp.float32),
    )(x, SW1, SB1.reshape(1, -1), SW3, SB3.reshape(1, -1),
      SW2, SB2.reshape(1, -1))

    out = pl.pallas_call(
        _combine_kernel,
        grid=(n_sb,),
        in_specs=[
            pl.BlockSpec((SB, DIM), lambda i: (i, 0)),
            pl.BlockSpec((TOPK * SB, DIM), lambda i: (i, 0)),
        ],
        out_specs=pl.BlockSpec((SB, DIM), lambda i: (i, 0)),
        out_shape=jax.ShapeDtypeStruct((TOK, DIM), jnp.float32),
    )(zx, y2)
    return out


# speedup vs baseline: 1.2561x; 1.0145x over previous
import functools

import jax
import jax.numpy as jnp
from jax.experimental import pallas as pl
from jax.experimental.pallas import tpu as pltpu
from jax.experimental.pallas import tpu_sc as plsc

TOPK = 2


def _sc_row_gather(table, idx):
    B = idx.shape[0]
    _, D = table.shape
    info = plsc.get_sparse_core_info()
    NW = info.num_cores * info.num_subcores
    b_per_w = B // NW
    CH = min(64, b_per_w)
    n_ch = b_per_w // CH
    mesh = plsc.VectorSubcoreMesh(core_axis_name="c", subcore_axis_name="s")

    @functools.partial(
        pl.kernel, mesh=mesh,
        out_type=jax.ShapeDtypeStruct((B, D), table.dtype),
        scratch_types=[
            pltpu.VMEM((2, CH), jnp.int32),
            pltpu.VMEM((2, CH, D), table.dtype),
            pltpu.SemaphoreType.DMA,
            pltpu.SemaphoreType.DMA,
        ],
    )
    def k(table_hbm, idx_hbm, out_hbm, idx_v, rows_v, sem0, sem1):
        wid = jax.lax.axis_index("s") * info.num_cores + jax.lax.axis_index("c")
        base = wid * b_per_w
        sems = (sem0, sem1)
        pltpu.sync_copy(idx_hbm.at[pl.ds(base, CH)], idx_v.at[0])
        pending = [pltpu.async_copy(table_hbm.at[idx_v.at[0]], rows_v.at[0],
                                    sems[0])]
        for c in range(n_ch):
            cur = c % 2
            if c + 1 < n_ch:
                nxt = (c + 1) % 2
                o2 = base + (c + 1) * CH
                pltpu.sync_copy(idx_hbm.at[pl.ds(o2, CH)], idx_v.at[nxt])
                pending.append(pltpu.async_copy(table_hbm.at[idx_v.at[nxt]],
                                                rows_v.at[nxt], sems[nxt]))
            pending[c].wait()
            pltpu.sync_copy(rows_v.at[cur], out_hbm.at[pl.ds(base + c * CH, CH)])

    return k(table, idx)


def _gate_kernel(x_ref, gwT_ref, gb_ref, i1_ref, i2_ref, w1_ref, w2_ref):
    xb = x_ref[...]
    gb = gb_ref[0, :]
    logits = jnp.dot(xb, gwT_ref[...], preferred_element_type=jnp.float32)
    logits = logits + gb[None, :]
    orig = jax.nn.sigmoid(logits)
    scores = orig + gb[None, :]
    E = scores.shape[1]
    iotaE = jax.lax.broadcasted_iota(jnp.int32, scores.shape, 1)
    m1 = jnp.max(scores, axis=1, keepdims=True)
    i1 = jnp.min(jnp.where(scores == m1, iotaE, E), axis=1)
    oh1 = iotaE == i1[:, None]
    v1 = jnp.sum(jnp.where(oh1, orig, 0.0), axis=1)
    scores2 = jnp.where(oh1, -jnp.inf, scores)
    m2 = jnp.max(scores2, axis=1, keepdims=True)
    i2 = jnp.min(jnp.where(scores2 == m2, iotaE, E), axis=1)
    oh2 = iotaE == i2[:, None]
    v2 = jnp.sum(jnp.where(oh2, orig, 0.0), axis=1)
    s = v1 + v2
    i1_ref[0, 0, :] = i1.astype(jnp.int32)
    i2_ref[0, 0, :] = i2.astype(jnp.int32)
    w1_ref[0, 0, :] = v1 / s
    w2_ref[0, 0, :] = v2 / s


def _moe_kernel(bids_ref, gids_ref, rs_ref, re_ref, init_ref,
                xs_ref, w_ref, W1_ref, B1_ref, W3_ref, B3_ref, W2_ref, B2_ref,
                out_ref, *, blk):
    t = pl.program_id(0)
    rs = rs_ref[t]
    re_ = re_ref[t]

    @pl.when(re_ > rs)
    def _():
        xb = xs_ref[...].astype(jnp.bfloat16)
        gr = bids_ref[t] * blk + jax.lax.broadcasted_iota(jnp.int32, (blk, 1), 0)
        mask = (gr >= rs) & (gr < re_)
        wv = w_ref[0, 0, :].reshape(blk, 1) * mask.astype(jnp.float32)
        h1 = jnp.dot(xb, W1_ref[0].astype(jnp.bfloat16),
                     preferred_element_type=jnp.float32) + B1_ref[0]
        h3 = jnp.dot(xb, W3_ref[0].astype(jnp.bfloat16),
                     preferred_element_type=jnp.float32) + B3_ref[0]
        h = (jax.nn.silu(h1) * h3 * wv).astype(jnp.bfloat16)
        contrib = jnp.dot(h, W2_ref[0].astype(jnp.bfloat16),
                          preferred_element_type=jnp.float32)
        contrib = contrib + wv * B2_ref[0]

        @pl.when(init_ref[t] == 1)
        def _():
            out_ref[...] = contrib

        @pl.when(init_ref[t] == 0)
        def _():
            out_ref[...] += contrib


def _shared_kernel(x_ref, sw1_ref, sb1_ref, sw3_ref, sb3_ref,
                   sw2_ref, sb2_ref, o_ref):
    xb = x_ref[...]
    xb16 = xb.astype(jnp.bfloat16)
    h1 = jnp.dot(xb16, sw1_ref[...].astype(jnp.bfloat16),
                 preferred_element_type=jnp.float32) + sb1_ref[0, :][None, :]
    h3 = jnp.dot(xb16, sw3_ref[...].astype(jnp.bfloat16),
                 preferred_element_type=jnp.float32) + sb3_ref[0, :][None, :]
    h = (jax.nn.silu(h1) * h3).astype(jnp.bfloat16)
    z = jnp.dot(h, sw2_ref[...].astype(jnp.bfloat16),
                preferred_element_type=jnp.float32) + sb2_ref[0, :][None, :]
    o_ref[...] = z + xb


def _combine_kernel(zx_ref, y2_ref, o_ref):
    sb, d = o_ref.shape
    y2b = y2_ref[...].reshape(sb, 2, d).astype(jnp.float32)
    o_ref[...] = zx_ref[...] + y2b[:, 0, :] + y2b[:, 1, :]


def kernel(x, gate_w, gate_b, W1, B1, W2, B2, W3, B3, SW1, SB1, SW2, SB2, SW3, SB3):
    TOK, DIM = x.shape
    E, _, INTER = W1.shape
    TOKF = TOK * TOPK
    BLK = 256
    NB = TOKF // BLK
    NT = NB + E
    GB = min(2048, TOK)
    SB = min(1024, TOK)

    n_gb = TOK // GB
    gwT = gate_w.T
    gb2 = gate_b.reshape(1, E)
    i1, i2, w1, w2 = pl.pallas_call(
        _gate_kernel,
        grid=(n_gb,),
        in_specs=[
            pl.BlockSpec((GB, DIM), lambda i: (i, 0)),
            pl.BlockSpec((DIM, E), lambda i: (0, 0)),
            pl.BlockSpec((1, E), lambda i: (0, 0)),
        ],
        out_specs=[
            pl.BlockSpec((1, 1, GB), lambda i: (i, 0, 0)),
            pl.BlockSpec((1, 1, GB), lambda i: (i, 0, 0)),
            pl.BlockSpec((1, 1, GB), lambda i: (i, 0, 0)),
            pl.BlockSpec((1, 1, GB), lambda i: (i, 0, 0)),
        ],
        out_shape=[
            jax.ShapeDtypeStruct((n_gb, 1, GB), jnp.int32),
            jax.ShapeDtypeStruct((n_gb, 1, GB), jnp.int32),
            jax.ShapeDtypeStruct((n_gb, 1, GB), jnp.float32),
            jax.ShapeDtypeStruct((n_gb, 1, GB), jnp.float32),
        ],
    )(x, gwT, gb2)
    i1 = i1.reshape(TOK)
    i2 = i2.reshape(TOK)
    w1 = w1.reshape(TOK)
    w2 = w2.reshape(TOK)

    e_flat = jnp.stack([i1, i2], axis=1).reshape(-1)
    w_flat = jnp.stack([w1, w2], axis=1).reshape(-1)

    flat_ids = jnp.arange(TOKF, dtype=jnp.int32)
    order = jnp.argsort(e_flat).astype(jnp.int32)
    sorted_tok = order // TOPK
    sorted_w = jnp.take(w_flat, order)
    del flat_ids
    pos = jnp.argsort(order).astype(jnp.int32)
    counts = jnp.bincount(e_flat, length=E).astype(jnp.int32)
    off = jnp.concatenate([jnp.zeros(1, jnp.int32),
                           jnp.cumsum(counts)]).astype(jnp.int32)
    xs = _sc_row_gather(x, sorted_tok)

    fb = off[:-1] // BLK
    lb = jnp.where(counts > 0, (off[1:] - 1) // BLK, 0)
    tiles_per = jnp.where(counts > 0, lb - fb + 1, 0).astype(jnp.int32)
    tile_off = jnp.concatenate([jnp.zeros(1, jnp.int32),
                                jnp.cumsum(tiles_per)]).astype(jnp.int32)
    T = tile_off[-1]
    gids = jnp.repeat(jnp.arange(E, dtype=jnp.int32), tiles_per,
                      total_repeat_length=NT)
    valid = jnp.arange(NT, dtype=jnp.int32) < T
    last_gid = jnp.take(gids, T - 1)
    gids = jnp.where(valid, gids, last_gid)
    j = jnp.arange(NT, dtype=jnp.int32) - tile_off[gids]
    bids = jnp.where(valid, fb[gids] + j, NB - 1).astype(jnp.int32)
    row_start = jnp.where(valid, off[gids], 0).astype(jnp.int32)
    row_end = jnp.where(valid, off[gids + 1], 0).astype(jnp.int32)
    init = jnp.concatenate([jnp.ones(1, jnp.int32),
                            (bids[1:] != bids[:-1]).astype(jnp.int32)])

    sw3d = sorted_w.reshape(NB, 1, BLK)

    grid_spec = pltpu.PrefetchScalarGridSpec(
        num_scalar_prefetch=5,
        grid=(NT,),
        in_specs=[
            pl.BlockSpec((BLK, DIM), lambda t, b, g, rs, re, it: (b[t], 0)),
            pl.BlockSpec((1, 1, BLK), lambda t, b, g, rs, re, it: (b[t], 0, 0)),
            pl.BlockSpec((1, DIM, INTER), lambda t, b, g, rs, re, it: (g[t], 0, 0)),
            pl.BlockSpec((1, 1, INTER), lambda t, b, g, rs, re, it: (g[t], 0, 0)),
            pl.BlockSpec((1, DIM, INTER), lambda t, b, g, rs, re, it: (g[t], 0, 0)),
            pl.BlockSpec((1, 1, INTER), lambda t, b, g, rs, re, it: (g[t], 0, 0)),
            pl.BlockSpec((1, INTER, DIM), lambda t, b, g, rs, re, it: (g[t], 0, 0)),
            pl.BlockSpec((1, 1, DIM), lambda t, b, g, rs, re, it: (g[t], 0, 0)),
        ],
        out_specs=pl.BlockSpec((BLK, DIM), lambda t, b, g, rs, re, it: (b[t], 0)),
    )
    outs = pl.pallas_call(
        functools.partial(_moe_kernel, blk=BLK),
        grid_spec=grid_spec,
        out_shape=jax.ShapeDtypeStruct((TOKF, DIM), jnp.float32),
    )(bids, gids, row_start, row_end, init,
      xs, sw3d, W1, B1.reshape(E, 1, INTER), W3, B3.reshape(E, 1, INTER),
      W2, B2.reshape(E, 1, DIM))

    y2 = _sc_row_gather(outs, pos)

    n_sb = TOK // SB
    zx = pl.pallas_call(
        _shared_kernel,
        grid=(n_sb,),
        in_specs=[
            pl.BlockSpec((SB, DIM), lambda i: (i, 0)),
            pl.BlockSpec((DIM, SW1.shape[1]), lambda i: (0, 0)),
            pl.BlockSpec((1, SW1.shape[1]), lambda i: (0, 0)),
            pl.BlockSpec((DIM, SW1.shape[1]), lambda i: (0, 0)),
            pl.BlockSpec((1, SW1.shape[1]), lambda i: (0, 0)),
            pl.BlockSpec((SW1.shape[1], DIM), lambda i: (0, 0)),
            pl.BlockSpec((1, DIM), lambda i: (0, 0)),
        ],
        out_specs=pl.BlockSpec((SB, DIM), lambda i: (i, 0)),
        out_shape=jax.ShapeDtypeStruct((TOK, DIM), jnp.float32),
    )(x, SW1, SB1.reshape(1, -1), SW3, SB3.reshape(1, -1),
      SW2, SB2.reshape(1, -1))

    out = pl.pallas_call(
        _combine_kernel,
        grid=(n_sb,),
        in_specs=[
            pl.BlockSpec((SB, DIM), lambda i: (i, 0)),
            pl.BlockSpec((TOPK * SB, DIM), lambda i: (i, 0)),
        ],
        out_specs=pl.BlockSpec((SB, DIM), lambda i: (i, 0)),
        out_shape=jax.ShapeDtypeStruct((TOK, DIM), jnp.float32),
    )(zx, y2)
    return out
